# baseline (speedup 1.0000x reference)
import functools

import jax
import jax.numpy as jnp
from jax import lax
from jax.experimental import pallas as pl
from jax.experimental.pallas import tpu as pltpu
from jax.experimental.pallas import tpu_sc as plsc

NC = 2
NS = 16
L = 16
K = 80
H = K // 2


def _chunk_block(nchunks):
    for cb in (25, 13, 7, 4, 1):
        if nchunks % cb == 0:
            return cb


@functools.lru_cache(maxsize=None)
def _make_sc_kernel(N, D, E):
    assert E % (NC * NS) == 0
    epw = E // (NC * NS)
    assert epw % K == 0
    nchunks = epw // K
    cb = _chunk_block(nchunks)
    nsc = nchunks // cb
    assert cb >= 4 and (cb - 4) % 3 == 0
    rpt = ((N + NS - 1) // NS + 7) // 8 * 8
    npad = rpt * NS
    nd = D // L

    mesh = plsc.VectorSubcoreMesh(core_axis_name="c", subcore_axis_name="s")

    @functools.partial(
        pl.kernel,
        out_type=jax.ShapeDtypeStruct((NC, npad, D), jnp.float32),
        mesh=mesh,
        scratch_types=[
            pltpu.VMEM((cb, K), jnp.int32),
            pltpu.VMEM((cb, 2, H), jnp.int32),
            pltpu.VMEM((cb, K), jnp.float32),
            pltpu.VMEM((K, D), jnp.float32),
            pltpu.VMEM((K, D), jnp.float32),
            pltpu.VMEM((K, D), jnp.float32),
            pltpu.VMEM_SHARED((npad, D), jnp.float32),
            pltpu.SemaphoreType.DMA,
            pltpu.SemaphoreType.DMA,
            pltpu.SemaphoreType.DMA,
            pltpu.SemaphoreType.DMA,
            pltpu.SemaphoreType.DMA,
            pltpu.SemaphoreType.DMA,
            pltpu.SemaphoreType.DMA,
        ],
    )
    def sc(x_hbm, idx_hbm, idxh_hbm, val_hbm, out_hbm,
           srcb, dstb, valb, r0, r1, r2, acc,
           g0, g1, g2, s0, s1, s2, sem_i):
        c = lax.axis_index("c")
        s = lax.axis_index("s")
        wid = c * NS + s
        rbuf = (r0, r1, r2)
        gsem = (g0, g1, g2)
        ssem = (s0, s1, s2)

        def preload_fire(j):
            pltpu.async_copy(idx_hbm.at[1, wid, j], srcb, sem_i)
            pltpu.async_copy(idxh_hbm.at[0, wid, j], dstb, sem_i)
            pltpu.async_copy(val_hbm.at[wid, j], valb, sem_i)

        def preload_wait(j):
            pltpu.make_async_copy(idx_hbm.at[1, wid, j], srcb, sem_i).wait()
            pltpu.make_async_copy(idxh_hbm.at[0, wid, j], dstb, sem_i).wait()
            pltpu.make_async_copy(val_hbm.at[wid, j], valb, sem_i).wait()

        preload_fire(0)

        zeros = jnp.zeros((L,), jnp.float32)

        def zrow(r, zcarry):
            for dd in range(nd):
                r0[r, pl.ds(dd * L, L)] = zeros
            return zcarry

        lax.fori_loop(0, K, zrow, 0)
        base = s * rpt
        nfull, rem = divmod(rpt, K)
        for t in range(nfull):
            pltpu.sync_copy(r0, acc.at[pl.ds(base + t * K, K)])
        if rem:
            pltpu.sync_copy(r0.at[pl.ds(0, rem)],
                            acc.at[pl.ds(base + nfull * K, rem)])
        plsc.subcore_barrier()

        def gfire(ci, b):
            pltpu.async_copy(x_hbm.at[srcb.at[ci]], rbuf[b], gsem[b])

        def gwait(b):
            pltpu.make_async_copy(
                x_hbm.at[srcb.at[0]], rbuf[b], gsem[b]).wait()

        def sfire_h(ci, b, h):
            pltpu.async_copy(rbuf[b].at[pl.ds(h * H, H)],
                             acc.at[dstb.at[ci, h]], ssem[b], add=True)

        def swait(b):
            for h in (0, 1):
                pltpu.make_async_copy(rbuf[b].at[pl.ds(h * H, H)],
                                      acc.at[dstb.at[0, h]], ssem[b]).wait()

        def scale_part(ci, b, lo, hi):
            buf = rbuf[b]

            def vec_body(j, rcarry):
                vals16 = valb[ci, pl.ds(j * L, L)]
                for t in range(L):
                    v = vals16[t]
                    r = j * L + t
                    for dd in range(nd):
                        buf[r, pl.ds(dd * L, L)] = (
                            buf[r, pl.ds(dd * L, L)] * v)
                return rcarry

            lax.fori_loop(lo, hi, vec_body, 0)

        ng = K // L
        nh = (H + L - 1) // L

        def chunk_work(ci, b, nxt):
            gwait(b)
            scale_part(ci, b, 0, nh)
            sfire_h(ci, b, 0)
            scale_part(ci, b, nh, ng)
            if nxt is not None:
                swait((b + 2) % 3)
                gfire(nxt, (b + 2) % 3)
            sfire_h(ci, b, 1)

        nmid = (cb - 4) // 3

        def block_body(j, bcarry):
            preload_wait(j)

            gfire(0, 0)
            gfire(1, 1)
            gfire(2, 2)
            chunk_work(0, 0, None)

            def mid_body(i, mcarry):
                ci = 3 * i + 1
                for b in (1, 2, 0):
                    chunk_work(ci, b, ci + 2)
                    ci = ci + 1
                return mcarry

            lax.fori_loop(0, nmid, mid_body, 0)

            chunk_work(cb - 3, 1, cb - 1)
            chunk_work(cb - 2, 2, None)
            chunk_work(cb - 1, 0, None)
            swait(1)
            swait(2)
            swait(0)

            @pl.when(j + 1 < nsc)
            def _():
                preload_fire(j + 1)
            return bcarry

        lax.fori_loop(0, nsc, block_body, 0)

        plsc.subcore_barrier()
        pltpu.sync_copy(acc.at[pl.ds(s * rpt, rpt)],
                        out_hbm.at[c, pl.ds(s * rpt, rpt)])

    return sc


@functools.lru_cache(maxsize=None)
def _make_combine(N, D):
    BR = 400
    assert N % BR == 0

    def body(p_ref, o_ref):
        o_ref[...] = p_ref[0] + p_ref[1]

    return pl.pallas_call(
        body,
        out_shape=jax.ShapeDtypeStruct((N, D), jnp.float32),
        grid=(N // BR,),
        in_specs=[pl.BlockSpec((2, BR, D), lambda i: (0, i, 0))],
        out_specs=pl.BlockSpec((BR, D), lambda i: (i, 0)),
    )


def kernel(x_source, neighborhood_indices, neighborhood_values):
    N, D = x_source.shape
    E = neighborhood_values.shape[0]
    epw = E // (NC * NS)
    nchunks = epw // K
    cb = _chunk_block(nchunks)
    shape = (NC * NS, nchunks // cb, cb, K)
    idx = neighborhood_indices.reshape((2,) + shape)
    idxh = neighborhood_indices.reshape((2,) + shape[:-1] + (2, H))
    val = neighborhood_values.reshape(shape)
    partials = _make_sc_kernel(N, D, E)(x_source, idx, idxh, val)
    return _make_combine(N, D)(partials)


# speedup vs baseline: 1.1297x; 1.1297x over previous
import functools

import jax
import jax.numpy as jnp
from jax import lax
from jax.experimental import pallas as pl
from jax.experimental.pallas import tpu as pltpu
from jax.experimental.pallas import tpu_sc as plsc

NC = 2
NS = 16
L = 16
K = 80
H = K // 2


def _chunk_block(nchunks):
    for cb in (25, 13, 7, 4, 1):
        if nchunks % cb == 0:
            return cb


@functools.lru_cache(maxsize=None)
def _make_sc_kernel(N, D, E):
    assert E % (NC * NS) == 0
    epw = E // (NC * NS)
    assert epw % K == 0
    nchunks = epw // K
    cb = _chunk_block(nchunks)
    nsc = nchunks // cb
    assert cb >= 4 and (cb - 4) % 3 == 0
    rpt = ((N + NS - 1) // NS + 7) // 8 * 8
    npad = rpt * NS
    nd = D // L

    mesh = plsc.VectorSubcoreMesh(core_axis_name="c", subcore_axis_name="s")

    @functools.partial(
        pl.kernel,
        out_type=jax.ShapeDtypeStruct((NC, npad, D), jnp.float32),
        mesh=mesh,
        scratch_types=[
            pltpu.VMEM((cb, K), jnp.int32),
            pltpu.VMEM((cb, K), jnp.int32),
            pltpu.VMEM((cb, K), jnp.float32),
            pltpu.VMEM((K, D), jnp.float32),
            pltpu.VMEM((K, D), jnp.float32),
            pltpu.VMEM((K, D), jnp.float32),
            pltpu.VMEM_SHARED((npad, D), jnp.float32),
            pltpu.SemaphoreType.DMA,
            pltpu.SemaphoreType.DMA,
            pltpu.SemaphoreType.DMA,
            pltpu.SemaphoreType.DMA,
            pltpu.SemaphoreType.DMA,
            pltpu.SemaphoreType.DMA,
            pltpu.SemaphoreType.DMA,
        ],
    )
    def sc(x_hbm, idx_hbm, val_hbm, out_hbm,
           srcb, dstb, valb, r0, r1, r2, acc,
           g0, g1, g2, s0, s1, s2, sem_i):
        c = lax.axis_index("c")
        s = lax.axis_index("s")
        wid = c * NS + s
        rbuf = (r0, r1, r2)
        gsem = (g0, g1, g2)
        ssem = (s0, s1, s2)

        def preload_fire(j):
            pltpu.async_copy(idx_hbm.at[1, wid, j], srcb, sem_i)
            pltpu.async_copy(idx_hbm.at[0, wid, j], dstb, sem_i)
            pltpu.async_copy(val_hbm.at[wid, j], valb, sem_i)

        def preload_wait(j):
            pltpu.make_async_copy(idx_hbm.at[1, wid, j], srcb, sem_i).wait()
            pltpu.make_async_copy(idx_hbm.at[0, wid, j], dstb, sem_i).wait()
            pltpu.make_async_copy(val_hbm.at[wid, j], valb, sem_i).wait()

        preload_fire(0)

        zeros = jnp.zeros((L,), jnp.float32)

        def zrow(r, zcarry):
            for dd in range(nd):
                r0[r, pl.ds(dd * L, L)] = zeros
            return zcarry

        lax.fori_loop(0, K, zrow, 0)
        base = s * rpt
        nfull, rem = divmod(rpt, K)
        for t in range(nfull):
            pltpu.sync_copy(r0, acc.at[pl.ds(base + t * K, K)])
        if rem:
            pltpu.sync_copy(r0.at[pl.ds(0, rem)],
                            acc.at[pl.ds(base + nfull * K, rem)])
        plsc.subcore_barrier()

        def gfire(ci, b):
            pltpu.async_copy(x_hbm.at[srcb.at[ci]], rbuf[b], gsem[b])

        def gwait(b):
            pltpu.make_async_copy(
                x_hbm.at[srcb.at[0]], rbuf[b], gsem[b]).wait()

        def sfire(ci, b):
            pltpu.async_copy(rbuf[b], acc.at[dstb.at[ci]], ssem[b],
                             add=True)

        def swait(b):
            pltpu.make_async_copy(
                rbuf[b], acc.at[dstb.at[0]], ssem[b]).wait()

        def scale(ci, b):
            buf = rbuf[b]

            def vec_body(j, rcarry):
                vals16 = valb[ci, pl.ds(j * L, L)]
                for t in range(L):
                    v = vals16[t]
                    r = j * L + t
                    for dd in range(nd):
                        buf[r, pl.ds(dd * L, L)] = (
                            buf[r, pl.ds(dd * L, L)] * v)
                return rcarry

            lax.fori_loop(0, K // L, vec_body, 0)

        nmid = (cb - 4) // 3

        def block_body(j, bcarry):
            preload_wait(j)

            gfire(0, 0)
            gfire(1, 1)
            gfire(2, 2)
            gwait(0)
            scale(0, 0)
            sfire(0, 0)

            def mid_body(i, mcarry):
                ci = 3 * i + 1
                for b in (1, 2, 0):
                    gwait(b)
                    scale(ci, b)
                    swait((b + 2) % 3)
                    gfire(ci + 2, (b + 2) % 3)
                    sfire(ci, b)
                    ci = ci + 1
                return mcarry

            lax.fori_loop(0, nmid, mid_body, 0)

            ci = cb - 3
            swait(0)
            gfire(cb - 1, 0)
            gwait(1)
            scale(ci, 1)
            sfire(ci, 1)

            swait(1)
            gwait(2)
            scale(ci + 1, 2)
            sfire(ci + 1, 2)

            swait(2)
            gwait(0)
            scale(ci + 2, 0)
            sfire(ci + 2, 0)
            swait(0)

            @pl.when(j + 1 < nsc)
            def _():
                preload_fire(j + 1)
            return bcarry

        lax.fori_loop(0, nsc, block_body, 0)

        plsc.subcore_barrier()
        pltpu.sync_copy(acc.at[pl.ds(s * rpt, rpt)],
                        out_hbm.at[c, pl.ds(s * rpt, rpt)])

    return sc


@functools.lru_cache(maxsize=None)
def _make_combine(N, D):
    BR = 400
    assert N % BR == 0

    def body(p_ref, o_ref):
        o_ref[...] = p_ref[0] + p_ref[1]

    return pl.pallas_call(
        body,
        out_shape=jax.ShapeDtypeStruct((N, D), jnp.float32),
        grid=(N // BR,),
        in_specs=[pl.BlockSpec((2, BR, D), lambda i: (0, i, 0))],
        out_specs=pl.BlockSpec((BR, D), lambda i: (i, 0)),
    )


def kernel(x_source, neighborhood_indices, neighborhood_values):
    N, D = x_source.shape
    E = neighborhood_values.shape[0]
    epw = E // (NC * NS)
    nchunks = epw // K
    cb = _chunk_block(nchunks)
    shape = (NC * NS, nchunks // cb, cb, K)
    idx = neighborhood_indices.reshape((2,) + shape)
    val = neighborhood_values.reshape(shape)
    partials = _make_sc_kernel(N, D, E)(x_source, idx, val)
    return _make_combine(N, D)(partials)
